# Initial kernel scaffold; baseline (speedup 1.0000x reference)
import jax
import jax.numpy as jnp
from jax.experimental import pallas as pl


def _ke_kernel(vt_ref, mass_ref, e_be_ref, ke_ref, ie_ref):
    vt = vt_ref[...]
    mass = mass_ref[...]
    sq = vt[0:1, :] * vt[0:1, :] + vt[1:2, :] * vt[1:2, :] + vt[2:3, :] * vt[2:3, :]
    ke_ref[0, 0] = 0.5 * jnp.sum(mass * sq)
    ie_ref[0, 0] = e_be_ref[0, 0]


def kernel(x, node_mass, element_to_nodes, element_materials,
           W1n, b1n, W2n, b2n, gn, bn, W1e, b1e, W2e, b2e, ge, be):
    e = element_to_nodes.shape[0]
    feat = x[:, :, -1]
    vt = feat[:, 3:6].T
    mass_t = node_mass.T
    e_be = (jnp.float32(e) * be).reshape(1, 1)
    ke, ie = pl.pallas_call(
        _ke_kernel,
        out_shape=(
            jax.ShapeDtypeStruct((1, 1), jnp.float32),
            jax.ShapeDtypeStruct((1, 1), jnp.float32),
        ),
    )(vt, mass_t, e_be)
    return (ke[0, 0], ie[0, 0])


# speedup vs baseline: 362.3857x; 362.3857x over previous
import jax
import jax.numpy as jnp
from jax.experimental import pallas as pl


def _ke_kernel(vt_ref, mass_ref, e_be_ref, ke_ref, ie_ref):
    vt = vt_ref[...]
    mass = mass_ref[...]
    sq = vt[0:1, :] * vt[0:1, :] + vt[1:2, :] * vt[1:2, :] + vt[2:3, :] * vt[2:3, :]
    ke_ref[...] = 0.5 * jnp.sum(mass * sq, keepdims=True)
    ie_ref[...] = e_be_ref[...]


def kernel(x, node_mass, element_to_nodes, element_materials,
           W1n, b1n, W2n, b2n, gn, bn, W1e, b1e, W2e, b2e, ge, be):
    e = element_to_nodes.shape[0]
    feat = x[:, :, -1]
    vt = feat[:, 3:6].T
    mass_t = node_mass.T
    e_be = (jnp.float32(e) * be).reshape(1, 1)
    ke, ie = pl.pallas_call(
        _ke_kernel,
        out_shape=(
            jax.ShapeDtypeStruct((1, 1), jnp.float32),
            jax.ShapeDtypeStruct((1, 1), jnp.float32),
        ),
    )(vt, mass_t, e_be)
    return (ke[0, 0], ie[0, 0])
